# baseline (speedup 1.0000x reference)
import functools

import jax
import jax.numpy as jnp
from jax import lax
from jax.experimental import pallas as pl
from jax.experimental.pallas import tpu as pltpu
from jax.experimental.pallas import tpu_sc as plsc

TILE = 256
SUB = 64


def _row_gather_call(table, idx, n_out, d):
    info = plsc.get_sparse_core_info()
    nc, ns = info.num_cores, info.num_subcores
    nw = nc * ns
    bpw = n_out // nw
    mesh = plsc.VectorSubcoreMesh(core_axis_name="c", subcore_axis_name="s")

    @functools.partial(
        pl.kernel,
        mesh=mesh,
        out_type=jax.ShapeDtypeStruct((n_out, d), jnp.float32),
        scratch_types=[
            pltpu.VMEM((bpw,), jnp.int32),
            pltpu.VMEM((bpw, d), jnp.float32),
            pltpu.SemaphoreType.DMA,
        ],
    )
    def gather_k(table_hbm, idx_hbm, out_hbm, idx_v, rows_v, sem):
        wid = lax.axis_index("s") * nc + lax.axis_index("c")
        base = wid * bpw
        pltpu.sync_copy(idx_hbm.at[pl.ds(base, bpw)], idx_v)
        pltpu.async_copy(table_hbm.at[idx_v], rows_v, sem).wait()
        pltpu.sync_copy(rows_v, out_hbm.at[pl.ds(base, bpw)])

    return gather_k(table, idx)


def _row_scatter_call(values, idx, n_out, d):
    n_in = values.shape[0]
    info = plsc.get_sparse_core_info()
    nc, ns = info.num_cores, info.num_subcores
    nw = nc * ns
    bpw = n_in // nw
    mesh = plsc.VectorSubcoreMesh(core_axis_name="c", subcore_axis_name="s")

    @functools.partial(
        pl.kernel,
        mesh=mesh,
        out_type=jax.ShapeDtypeStruct((n_out, d), values.dtype),
        scratch_types=[
            pltpu.VMEM((bpw,), jnp.int32),
            pltpu.VMEM((bpw, d), values.dtype),
            pltpu.SemaphoreType.DMA,
        ],
    )
    def scatter_k(vals_hbm, idx_hbm, out_hbm, idx_v, rows_v, sem):
        wid = lax.axis_index("s") * nc + lax.axis_index("c")
        base = wid * bpw
        pltpu.sync_copy(idx_hbm.at[pl.ds(base, bpw)], idx_v)
        pltpu.sync_copy(vals_hbm.at[pl.ds(base, bpw)], rows_v)
        pltpu.async_copy(rows_v, out_hbm.at[idx_v], sem).wait()

    return scatter_k(values, idx)


def _matmul_tile_kernel(texp_ref, x_ref, w_ref, b_ref, o_ref):
    i = pl.program_id(0)
    n_sub_per_tile = TILE // SUB
    for j in range(n_sub_per_tile):
        e = texp_ref[n_sub_per_tile * i + j]
        o_ref[j * SUB:(j + 1) * SUB, :] = (
            jnp.dot(
                x_ref[j * SUB:(j + 1) * SUB, :],
                w_ref[e],
                preferred_element_type=jnp.float32,
            )
            + b_ref[e, 0]
        )


def kernel(inputs, index, weight, bias):
    tokens, in_f = inputs.shape
    n_exp, _, out_f = weight.shape

    import math as _math

    align = TILE * 256 // _math.gcd(TILE, 256)
    p_rows = tokens + n_exp * (SUB - 1)
    p_rows = ((p_rows + align - 1) // align) * align
    n_tiles = p_rows // TILE
    n_subs = p_rows // SUB

    idx = index.astype(jnp.int32)
    oh = (idx[:, None] == jnp.arange(n_exp, dtype=jnp.int32)[None, :]).astype(
        jnp.int32
    )
    ccum = jnp.cumsum(oh, axis=0)
    counts = ccum[-1]
    subs_per_e = (counts + SUB - 1) // SUB
    sub_end = jnp.cumsum(subs_per_e)
    row_start = (sub_end - subs_per_e) * SUB
    pos = jnp.sum(oh * (ccum - 1 + row_start[None, :]), axis=1)
    sub_id = jnp.arange(n_subs, dtype=jnp.int32)
    sub_expert = jnp.minimum(
        jnp.sum((sub_end[None, :] <= sub_id[:, None]).astype(jnp.int32), axis=1),
        n_exp - 1,
    ).astype(jnp.int32)

    x_sorted = _row_scatter_call(inputs, pos, p_rows, in_f)

    grid_spec = pltpu.PrefetchScalarGridSpec(
        num_scalar_prefetch=1,
        grid=(n_tiles,),
        in_specs=[
            pl.BlockSpec((TILE, in_f), lambda i, texp: (i, 0)),
            pl.BlockSpec((n_exp, in_f, out_f), lambda i, texp: (0, 0, 0)),
            pl.BlockSpec((n_exp, 1, out_f), lambda i, texp: (0, 0, 0)),
        ],
        out_specs=pl.BlockSpec((TILE, out_f), lambda i, texp: (i, 0)),
    )
    y_sorted = pl.pallas_call(
        _matmul_tile_kernel,
        grid_spec=grid_spec,
        out_shape=jax.ShapeDtypeStruct((p_rows, out_f), jnp.float32),
    )(sub_expert, x_sorted, weight, bias[:, None, :])

    return _row_gather_call(y_sorted, pos, tokens, out_f)


# speedup vs baseline: 1.0017x; 1.0017x over previous
import functools

import jax
import jax.numpy as jnp
from jax import lax
from jax.experimental import pallas as pl
from jax.experimental.pallas import tpu as pltpu
from jax.experimental.pallas import tpu_sc as plsc

TILE = 256
SUB = 128


def _row_gather_call(table, idx, n_out, d):
    info = plsc.get_sparse_core_info()
    nc, ns = info.num_cores, info.num_subcores
    nw = nc * ns
    bpw = n_out // nw
    mesh = plsc.VectorSubcoreMesh(core_axis_name="c", subcore_axis_name="s")

    @functools.partial(
        pl.kernel,
        mesh=mesh,
        out_type=jax.ShapeDtypeStruct((n_out, d), jnp.float32),
        scratch_types=[
            pltpu.VMEM((bpw,), jnp.int32),
            pltpu.VMEM((bpw, d), jnp.float32),
            pltpu.SemaphoreType.DMA,
        ],
    )
    def gather_k(table_hbm, idx_hbm, out_hbm, idx_v, rows_v, sem):
        wid = lax.axis_index("s") * nc + lax.axis_index("c")
        base = wid * bpw
        pltpu.sync_copy(idx_hbm.at[pl.ds(base, bpw)], idx_v)
        pltpu.async_copy(table_hbm.at[idx_v], rows_v, sem).wait()
        pltpu.sync_copy(rows_v, out_hbm.at[pl.ds(base, bpw)])

    return gather_k(table, idx)


def _row_scatter_call(values, idx, n_out, d):
    n_in = values.shape[0]
    info = plsc.get_sparse_core_info()
    nc, ns = info.num_cores, info.num_subcores
    nw = nc * ns
    bpw = n_in // nw
    mesh = plsc.VectorSubcoreMesh(core_axis_name="c", subcore_axis_name="s")

    @functools.partial(
        pl.kernel,
        mesh=mesh,
        out_type=jax.ShapeDtypeStruct((n_out, d), values.dtype),
        scratch_types=[
            pltpu.VMEM((bpw,), jnp.int32),
            pltpu.VMEM((bpw, d), values.dtype),
            pltpu.SemaphoreType.DMA,
        ],
    )
    def scatter_k(vals_hbm, idx_hbm, out_hbm, idx_v, rows_v, sem):
        wid = lax.axis_index("s") * nc + lax.axis_index("c")
        base = wid * bpw
        pltpu.sync_copy(idx_hbm.at[pl.ds(base, bpw)], idx_v)
        pltpu.sync_copy(vals_hbm.at[pl.ds(base, bpw)], rows_v)
        pltpu.async_copy(rows_v, out_hbm.at[idx_v], sem).wait()

    return scatter_k(values, idx)


def _matmul_tile_kernel(texp_ref, x_ref, w_ref, b_ref, o_ref):
    i = pl.program_id(0)
    n_sub_per_tile = TILE // SUB
    for j in range(n_sub_per_tile):
        e = texp_ref[n_sub_per_tile * i + j]
        o_ref[j * SUB:(j + 1) * SUB, :] = (
            jnp.dot(
                x_ref[j * SUB:(j + 1) * SUB, :],
                w_ref[e],
                preferred_element_type=jnp.float32,
            )
            + b_ref[e, 0]
        )


def kernel(inputs, index, weight, bias):
    tokens, in_f = inputs.shape
    n_exp, _, out_f = weight.shape

    import math as _math

    align = TILE * 256 // _math.gcd(TILE, 256)
    p_rows = tokens + n_exp * (SUB - 1)
    p_rows = ((p_rows + align - 1) // align) * align
    n_tiles = p_rows // TILE
    n_subs = p_rows // SUB

    idx = index.astype(jnp.int32)
    oh = (idx[:, None] == jnp.arange(n_exp, dtype=jnp.int32)[None, :]).astype(
        jnp.int32
    )
    ccum = jnp.cumsum(oh, axis=0)
    counts = ccum[-1]
    subs_per_e = (counts + SUB - 1) // SUB
    sub_end = jnp.cumsum(subs_per_e)
    row_start = (sub_end - subs_per_e) * SUB
    pos = jnp.sum(oh * (ccum - 1 + row_start[None, :]), axis=1)
    sub_id = jnp.arange(n_subs, dtype=jnp.int32)
    sub_expert = jnp.minimum(
        jnp.sum((sub_end[None, :] <= sub_id[:, None]).astype(jnp.int32), axis=1),
        n_exp - 1,
    ).astype(jnp.int32)

    x_sorted = _row_scatter_call(inputs, pos, p_rows, in_f)

    grid_spec = pltpu.PrefetchScalarGridSpec(
        num_scalar_prefetch=1,
        grid=(n_tiles,),
        in_specs=[
            pl.BlockSpec((TILE, in_f), lambda i, texp: (i, 0)),
            pl.BlockSpec((n_exp, in_f, out_f), lambda i, texp: (0, 0, 0)),
            pl.BlockSpec((n_exp, 1, out_f), lambda i, texp: (0, 0, 0)),
        ],
        out_specs=pl.BlockSpec((TILE, out_f), lambda i, texp: (i, 0)),
    )
    y_sorted = pl.pallas_call(
        _matmul_tile_kernel,
        grid_spec=grid_spec,
        out_shape=jax.ShapeDtypeStruct((p_rows, out_f), jnp.float32),
    )(sub_expert, x_sorted, weight, bias[:, None, :])

    return _row_gather_call(y_sorted, pos, tokens, out_f)


# speedup vs baseline: 1.0682x; 1.0664x over previous
import functools

import jax
import jax.numpy as jnp
from jax import lax
from jax.experimental import pallas as pl
from jax.experimental.pallas import tpu as pltpu
from jax.experimental.pallas import tpu_sc as plsc

TILE = 256
SUB = 128


def _row_gather_call(table, idx, n_out, d):
    info = plsc.get_sparse_core_info()
    nc, ns = info.num_cores, info.num_subcores
    nw = nc * ns
    bpw = n_out // nw
    mesh = plsc.VectorSubcoreMesh(core_axis_name="c", subcore_axis_name="s")

    @functools.partial(
        pl.kernel,
        mesh=mesh,
        out_type=jax.ShapeDtypeStruct((n_out, d), jnp.float32),
        scratch_types=[
            pltpu.VMEM((bpw,), jnp.int32),
            pltpu.VMEM((bpw, d), jnp.float32),
            pltpu.SemaphoreType.DMA,
        ],
    )
    def gather_k(table_hbm, idx_hbm, out_hbm, idx_v, rows_v, sem):
        wid = lax.axis_index("s") * nc + lax.axis_index("c")
        base = wid * bpw
        pltpu.sync_copy(idx_hbm.at[pl.ds(base, bpw)], idx_v)
        pltpu.async_copy(table_hbm.at[idx_v], rows_v, sem).wait()
        pltpu.sync_copy(rows_v, out_hbm.at[pl.ds(base, bpw)])

    return gather_k(table, idx)


def _row_scatter_call(values, idx, n_out, d):
    n_in = values.shape[0]
    info = plsc.get_sparse_core_info()
    nc, ns = info.num_cores, info.num_subcores
    nw = nc * ns
    bpw = n_in // nw
    mesh = plsc.VectorSubcoreMesh(core_axis_name="c", subcore_axis_name="s")

    @functools.partial(
        pl.kernel,
        mesh=mesh,
        out_type=jax.ShapeDtypeStruct((n_out, d), values.dtype),
        scratch_types=[
            pltpu.VMEM((bpw,), jnp.int32),
            pltpu.VMEM((bpw, d), values.dtype),
            pltpu.SemaphoreType.DMA,
        ],
    )
    def scatter_k(vals_hbm, idx_hbm, out_hbm, idx_v, rows_v, sem):
        wid = lax.axis_index("s") * nc + lax.axis_index("c")
        base = wid * bpw
        pltpu.sync_copy(idx_hbm.at[pl.ds(base, bpw)], idx_v)
        pltpu.sync_copy(vals_hbm.at[pl.ds(base, bpw)], rows_v)
        pltpu.async_copy(rows_v, out_hbm.at[idx_v], sem).wait()

    return scatter_k(values, idx)


_SL, _LN = 16, 128


def _meta_kernel(idx_ref, pos_ref, sexp_ref, n_exp, n_subs):
    idx = idx_ref[...]
    lane = jax.lax.broadcasted_iota(jnp.int32, (_SL, _LN), 1)
    subl = jax.lax.broadcasted_iota(jnp.int32, (_SL, _LN), 0)
    pos_acc = jnp.zeros((_SL, _LN), jnp.int32)
    sexp = jnp.zeros((1, _LN), jnp.int32)
    sub_end = jnp.int32(0)
    for e in range(n_exp):
        m = (idx == e).astype(jnp.int32)
        p = m
        k = 1
        while k < _LN:
            p = p + jnp.where(lane >= k, jnp.roll(p, k, axis=1), 0)
            k *= 2
        rt = jax.lax.broadcast_in_dim(p[:, _LN - 1], (_SL, _LN), (0,))
        q = rt
        k = 1
        while k < _SL:
            q = q + jnp.where(subl >= k, jnp.roll(q, k, axis=0), 0)
            k *= 2
        ccum = p + (q - rt)
        count_e = jnp.max(q)
        row_start_e = sub_end * SUB
        pos_acc = pos_acc + m * (ccum - 1 + row_start_e)
        sub_end = sub_end + (count_e + SUB - 1) // SUB
        sexp = sexp + jnp.where(lane[0:1, :] >= sub_end, 1, 0)
    pos_ref[...] = pos_acc
    sexp_ref[...] = jnp.minimum(sexp, n_exp - 1)


def _matmul_tile_kernel(texp_ref, x_ref, w_ref, b_ref, o_ref):
    i = pl.program_id(0)
    n_sub_per_tile = TILE // SUB
    for j in range(n_sub_per_tile):
        e = texp_ref[n_sub_per_tile * i + j]
        o_ref[j * SUB:(j + 1) * SUB, :] = (
            jnp.dot(
                x_ref[j * SUB:(j + 1) * SUB, :],
                w_ref[e],
                preferred_element_type=jnp.float32,
            )
            + b_ref[e, 0]
        )


def kernel(inputs, index, weight, bias):
    tokens, in_f = inputs.shape
    n_exp, _, out_f = weight.shape

    import math as _math

    align = TILE * 256 // _math.gcd(TILE, 256)
    p_rows = tokens + n_exp * (SUB - 1)
    p_rows = ((p_rows + align - 1) // align) * align
    n_tiles = p_rows // TILE
    n_subs = p_rows // SUB

    idx = index.astype(jnp.int32).reshape(_SL, _LN)
    pos2d, sexp_row = pl.pallas_call(
        functools.partial(_meta_kernel, n_exp=n_exp, n_subs=n_subs),
        out_shape=(
            jax.ShapeDtypeStruct((_SL, _LN), jnp.int32),
            jax.ShapeDtypeStruct((1, _LN), jnp.int32),
        ),
    )(idx)
    pos = pos2d.reshape(tokens)
    sub_expert = sexp_row[0, :n_subs]

    x_sorted = _row_scatter_call(inputs, pos, p_rows, in_f)

    grid_spec = pltpu.PrefetchScalarGridSpec(
        num_scalar_prefetch=1,
        grid=(n_tiles,),
        in_specs=[
            pl.BlockSpec((TILE, in_f), lambda i, texp: (i, 0)),
            pl.BlockSpec((n_exp, in_f, out_f), lambda i, texp: (0, 0, 0)),
            pl.BlockSpec((n_exp, 1, out_f), lambda i, texp: (0, 0, 0)),
        ],
        out_specs=pl.BlockSpec((TILE, out_f), lambda i, texp: (i, 0)),
    )
    y_sorted = pl.pallas_call(
        _matmul_tile_kernel,
        grid_spec=grid_spec,
        out_shape=jax.ShapeDtypeStruct((p_rows, out_f), jnp.float32),
    )(sub_expert, x_sorted, weight, bias[:, None, :])

    return _row_gather_call(y_sorted, pos, tokens, out_f)


# speedup vs baseline: 1.0792x; 1.0103x over previous
import functools

import jax
import jax.numpy as jnp
from jax import lax
from jax.experimental import pallas as pl
from jax.experimental.pallas import tpu as pltpu
from jax.experimental.pallas import tpu_sc as plsc

TILE = 256
SUB = 128


def _row_gather_call(table, idx, n_out, d):
    info = plsc.get_sparse_core_info()
    nc, ns = info.num_cores, info.num_subcores
    nw = nc * ns
    bpw = n_out // nw
    mesh = plsc.VectorSubcoreMesh(core_axis_name="c", subcore_axis_name="s")

    @functools.partial(
        pl.kernel,
        mesh=mesh,
        out_type=jax.ShapeDtypeStruct((n_out, d), jnp.float32),
        scratch_types=[
            pltpu.VMEM((bpw,), jnp.int32),
            pltpu.VMEM((bpw, d), jnp.float32),
            pltpu.SemaphoreType.DMA,
        ],
    )
    def gather_k(table_hbm, idx_hbm, out_hbm, idx_v, rows_v, sem):
        wid = lax.axis_index("s") * nc + lax.axis_index("c")
        base = wid * bpw
        pltpu.sync_copy(idx_hbm.at[pl.ds(base, bpw)], idx_v)
        pltpu.async_copy(table_hbm.at[idx_v], rows_v, sem).wait()
        pltpu.sync_copy(rows_v, out_hbm.at[pl.ds(base, bpw)])

    return gather_k(table, idx)


def _row_scatter_call(values, idx, n_out, d):
    n_in = values.shape[0]
    info = plsc.get_sparse_core_info()
    nc, ns = info.num_cores, info.num_subcores
    nw = nc * ns
    bpw = n_in // nw
    mesh = plsc.VectorSubcoreMesh(core_axis_name="c", subcore_axis_name="s")

    @functools.partial(
        pl.kernel,
        mesh=mesh,
        out_type=jax.ShapeDtypeStruct((n_out, d), values.dtype),
        scratch_types=[
            pltpu.VMEM((bpw,), jnp.int32),
            pltpu.VMEM((bpw, d), values.dtype),
            pltpu.SemaphoreType.DMA,
            pltpu.SemaphoreType.DMA,
        ],
    )
    def scatter_k(vals_hbm, idx_hbm, out_hbm, idx_v, rows_v, sem, sem2):
        wid = lax.axis_index("s") * nc + lax.axis_index("c")
        base = wid * bpw
        cp_idx = pltpu.async_copy(idx_hbm.at[pl.ds(base, bpw)], idx_v, sem2)
        cp_rows = pltpu.async_copy(vals_hbm.at[pl.ds(base, bpw)], rows_v, sem)
        cp_idx.wait()
        cp_rows.wait()
        pltpu.async_copy(rows_v, out_hbm.at[idx_v], sem).wait()

    return scatter_k(values, idx)


_SL, _LN = 16, 128


def _meta_kernel(idx_ref, pos_ref, sexp_ref, n_exp, n_subs):
    idx = idx_ref[...]
    lane = jax.lax.broadcasted_iota(jnp.int32, (_SL, _LN), 1)
    subl = jax.lax.broadcasted_iota(jnp.int32, (_SL, _LN), 0)
    pos_acc = jnp.zeros((_SL, _LN), jnp.int32)
    sexp = jnp.zeros((1, _LN), jnp.int32)
    sub_end = jnp.int32(0)
    for e in range(n_exp):
        m = (idx == e).astype(jnp.int32)
        p = m
        k = 1
        while k < _LN:
            p = p + jnp.where(lane >= k, jnp.roll(p, k, axis=1), 0)
            k *= 2
        rt = jax.lax.broadcast_in_dim(p[:, _LN - 1], (_SL, _LN), (0,))
        q = rt
        k = 1
        while k < _SL:
            q = q + jnp.where(subl >= k, jnp.roll(q, k, axis=0), 0)
            k *= 2
        ccum = p + (q - rt)
        count_e = jnp.max(q)
        row_start_e = sub_end * SUB
        pos_acc = pos_acc + m * (ccum - 1 + row_start_e)
        sub_end = sub_end + (count_e + SUB - 1) // SUB
        sexp = sexp + jnp.where(lane[0:1, :] >= sub_end, 1, 0)
    pos_ref[...] = pos_acc
    sexp_ref[...] = jnp.minimum(sexp, n_exp - 1)


def _matmul_tile_kernel(texp_ref, x_ref, w_ref, b_ref, o_ref):
    i = pl.program_id(0)
    n_sub_per_tile = TILE // SUB
    for j in range(n_sub_per_tile):
        e = texp_ref[0, n_sub_per_tile * i + j]
        o_ref[j * SUB:(j + 1) * SUB, :] = (
            jnp.dot(
                x_ref[j * SUB:(j + 1) * SUB, :],
                w_ref[e],
                preferred_element_type=jnp.float32,
            )
            + b_ref[e]
        )


def kernel(inputs, index, weight, bias):
    tokens, in_f = inputs.shape
    n_exp, _, out_f = weight.shape

    import math as _math

    align = TILE * 256 // _math.gcd(TILE, 256)
    p_rows = tokens + n_exp * (SUB - 1)
    p_rows = ((p_rows + align - 1) // align) * align
    n_tiles = p_rows // TILE
    n_subs = p_rows // SUB

    idx = index.astype(jnp.int32).reshape(_SL, _LN)
    pos2d, sexp_row = pl.pallas_call(
        functools.partial(_meta_kernel, n_exp=n_exp, n_subs=n_subs),
        out_shape=(
            jax.ShapeDtypeStruct((_SL, _LN), jnp.int32),
            jax.ShapeDtypeStruct((1, _LN), jnp.int32),
        ),
    )(idx)
    pos = pos2d.reshape(tokens)

    x_sorted = _row_scatter_call(inputs, pos, p_rows, in_f)

    grid_spec = pltpu.PrefetchScalarGridSpec(
        num_scalar_prefetch=1,
        grid=(n_tiles,),
        in_specs=[
            pl.BlockSpec((TILE, in_f), lambda i, texp: (i, 0)),
            pl.BlockSpec((n_exp, in_f, out_f), lambda i, texp: (0, 0, 0)),
            pl.BlockSpec((n_exp, out_f), lambda i, texp: (0, 0)),
        ],
        out_specs=pl.BlockSpec((TILE, out_f), lambda i, texp: (i, 0)),
    )
    y_sorted = pl.pallas_call(
        _matmul_tile_kernel,
        grid_spec=grid_spec,
        out_shape=jax.ShapeDtypeStruct((p_rows, out_f), jnp.float32),
    )(sexp_row, x_sorted, weight, bias)

    return _row_gather_call(y_sorted, pos, tokens, out_f)


# speedup vs baseline: 1.1292x; 1.0463x over previous
import functools

import jax
import jax.numpy as jnp
from jax import lax
from jax.experimental import pallas as pl
from jax.experimental.pallas import tpu as pltpu
from jax.experimental.pallas import tpu_sc as plsc

TILE = 384
SUB = 128


def _row_gather_call(table, idx, n_out, d):
    info = plsc.get_sparse_core_info()
    nc, ns = info.num_cores, info.num_subcores
    nw = nc * ns
    bpw = n_out // nw
    mesh = plsc.VectorSubcoreMesh(core_axis_name="c", subcore_axis_name="s")

    @functools.partial(
        pl.kernel,
        mesh=mesh,
        out_type=jax.ShapeDtypeStruct((n_out, d), jnp.float32),
        scratch_types=[
            pltpu.VMEM((bpw,), jnp.int32),
            pltpu.VMEM((bpw, d), jnp.float32),
            pltpu.SemaphoreType.DMA,
        ],
    )
    def gather_k(table_hbm, idx_hbm, out_hbm, idx_v, rows_v, sem):
        wid = lax.axis_index("s") * nc + lax.axis_index("c")
        base = wid * bpw
        pltpu.sync_copy(idx_hbm.at[pl.ds(base, bpw)], idx_v)
        pltpu.async_copy(table_hbm.at[idx_v], rows_v, sem).wait()
        pltpu.sync_copy(rows_v, out_hbm.at[pl.ds(base, bpw)])

    return gather_k(table, idx)


def _row_scatter_call(values, idx, n_out, d):
    n_in = values.shape[0]
    info = plsc.get_sparse_core_info()
    nc, ns = info.num_cores, info.num_subcores
    nw = nc * ns
    bpw = n_in // nw
    mesh = plsc.VectorSubcoreMesh(core_axis_name="c", subcore_axis_name="s")

    @functools.partial(
        pl.kernel,
        mesh=mesh,
        out_type=jax.ShapeDtypeStruct((n_out, d), values.dtype),
        scratch_types=[
            pltpu.VMEM((bpw,), jnp.int32),
            pltpu.VMEM((bpw, d), values.dtype),
            pltpu.SemaphoreType.DMA,
            pltpu.SemaphoreType.DMA,
        ],
    )
    def scatter_k(vals_hbm, idx_hbm, out_hbm, idx_v, rows_v, sem, sem2):
        wid = lax.axis_index("s") * nc + lax.axis_index("c")
        base = wid * bpw
        cp_idx = pltpu.async_copy(idx_hbm.at[pl.ds(base, bpw)], idx_v, sem2)
        cp_rows = pltpu.async_copy(vals_hbm.at[pl.ds(base, bpw)], rows_v, sem)
        cp_idx.wait()
        cp_rows.wait()
        pltpu.async_copy(rows_v, out_hbm.at[idx_v], sem).wait()

    return scatter_k(values, idx)


_SL, _LN = 16, 128


def _meta_kernel(idx_ref, pos_ref, sexp_ref, n_exp, n_subs):
    idx = idx_ref[...]
    lane = jax.lax.broadcasted_iota(jnp.int32, (_SL, _LN), 1)
    subl = jax.lax.broadcasted_iota(jnp.int32, (_SL, _LN), 0)
    pos_acc = jnp.zeros((_SL, _LN), jnp.int32)
    sexp = jnp.zeros((1, _LN), jnp.int32)
    sub_end = jnp.int32(0)
    for e in range(n_exp):
        m = (idx == e).astype(jnp.int32)
        p = m
        k = 1
        while k < _LN:
            p = p + jnp.where(lane >= k, jnp.roll(p, k, axis=1), 0)
            k *= 2
        rt = jax.lax.broadcast_in_dim(p[:, _LN - 1], (_SL, _LN), (0,))
        q = rt
        k = 1
        while k < _SL:
            q = q + jnp.where(subl >= k, jnp.roll(q, k, axis=0), 0)
            k *= 2
        ccum = p + (q - rt)
        count_e = jnp.max(q)
        row_start_e = sub_end * SUB
        pos_acc = pos_acc + m * (ccum - 1 + row_start_e)
        sub_end = sub_end + (count_e + SUB - 1) // SUB
        sexp = sexp + jnp.where(lane[0:1, :] >= sub_end, 1, 0)
    pos_ref[...] = pos_acc
    sexp_ref[...] = jnp.minimum(sexp, n_exp - 1)


def _matmul_tile_kernel(texp_ref, x_ref, w_ref, b_ref, o_ref):
    i = pl.program_id(0)
    n_sub_per_tile = TILE // SUB
    for j in range(n_sub_per_tile):
        e = texp_ref[0, n_sub_per_tile * i + j]
        o_ref[j * SUB:(j + 1) * SUB, :] = (
            jnp.dot(
                x_ref[j * SUB:(j + 1) * SUB, :],
                w_ref[e],
                preferred_element_type=jnp.float32,
            )
            + b_ref[e]
        )


def kernel(inputs, index, weight, bias):
    tokens, in_f = inputs.shape
    n_exp, _, out_f = weight.shape

    import math as _math

    align = TILE * 256 // _math.gcd(TILE, 256)
    p_rows = tokens + n_exp * (SUB - 1)
    p_rows = ((p_rows + align - 1) // align) * align
    n_tiles = p_rows // TILE
    n_subs = p_rows // SUB

    idx = index.astype(jnp.int32).reshape(_SL, _LN)
    pos2d, sexp_row = pl.pallas_call(
        functools.partial(_meta_kernel, n_exp=n_exp, n_subs=n_subs),
        out_shape=(
            jax.ShapeDtypeStruct((_SL, _LN), jnp.int32),
            jax.ShapeDtypeStruct((1, _LN), jnp.int32),
        ),
    )(idx)
    pos = pos2d.reshape(tokens)

    x_sorted = _row_scatter_call(inputs, pos, p_rows, in_f)

    grid_spec = pltpu.PrefetchScalarGridSpec(
        num_scalar_prefetch=1,
        grid=(n_tiles,),
        in_specs=[
            pl.BlockSpec((TILE, in_f), lambda i, texp: (i, 0)),
            pl.BlockSpec((n_exp, in_f, out_f), lambda i, texp: (0, 0, 0)),
            pl.BlockSpec((n_exp, out_f), lambda i, texp: (0, 0)),
        ],
        out_specs=pl.BlockSpec((TILE, out_f), lambda i, texp: (i, 0)),
    )
    y_sorted = pl.pallas_call(
        _matmul_tile_kernel,
        grid_spec=grid_spec,
        out_shape=jax.ShapeDtypeStruct((p_rows, out_f), jnp.float32),
    )(sexp_row, x_sorted, weight, bias)

    return _row_gather_call(y_sorted, pos, tokens, out_f)
